# baseline (speedup 1.0000x reference)
import functools

import jax
import jax.numpy as jnp
from jax import lax
from jax.experimental import pallas as pl
from jax.experimental.pallas import tpu as pltpu
from jax.experimental.pallas import tpu_sc as plsc

N = 10000
L = 256
LANES = 128
BE = 256
NTILES = 16
GH = 2048
NACC = GH + 16
ZPT = GH // 16
NG = -(-N // GH)
NP16 = N + 16
BEB = 1024


def _pad_edges(src, dst, n_edges, nworkers):
    step = nworkers * BE
    e_pad = step * ((n_edges + step - 1) // step)
    pad = e_pad - n_edges
    src_p = jnp.concatenate([src, jnp.zeros((pad,), jnp.int32)])
    dst_p = jnp.concatenate([dst, jnp.full((pad,), N, jnp.int32)])
    nit = e_pad // step
    return (src_p.reshape(nworkers, nit, BE), dst_p.reshape(nworkers, nit, BE),
            nit, e_pad)


def _mesh():
    return plsc.VectorSubcoreMesh(core_axis_name="c", subcore_axis_name="s")


def _make_scatter_add(nchunks, nit, stride):
    niter = (nchunks + 1) // 2

    def body(table, srct, dstt, zeros, out, src_v, dst_v, idx_v, didx_v,
             rows_v, acc_sh, sem):
        core = lax.axis_index("c")
        tile = lax.axis_index("s")
        pltpu.sync_copy(srct.at[tile], src_v)
        pltpu.sync_copy(dstt.at[tile], dst_v)
        for ci in range(niter):
            cid = 2 * ci + core
            if 2 * ci + 1 >= nchunks:
                cid = jnp.minimum(cid, nchunks - 1)
            if True:
                for g in range(NG):
                    base = g * GH
                    rows = min(GH, N - base)
                    wpt = (rows // (8 * NTILES)) * 8
                    plsc.subcore_barrier()
                    pltpu.sync_copy(zeros.at[pl.ds(tile * ZPT, ZPT)],
                                    acc_sh.at[pl.ds(tile * ZPT, ZPT)])

                    @pl.when(tile == 0)
                    def _zero_tail():
                        pltpu.sync_copy(
                            zeros.at[pl.ds(16 * ZPT, NACC - 16 * ZPT)],
                            acc_sh.at[pl.ds(16 * ZPT, NACC - 16 * ZPT)])

                    plsc.subcore_barrier()

                    @pl.loop(0, nit)
                    def _edge_step(it):
                        off = cid * stride
                        for j in range(BE // 16):
                            s16 = src_v[it, pl.ds(16 * j, 16)]
                            d16 = dst_v[it, pl.ds(16 * j, 16)] - base
                            ok = (d16 >= 0) & (d16 < rows)
                            idx_v[pl.ds(16 * j, 16)] = s16 + off
                            didx_v[pl.ds(16 * j, 16)] = jnp.where(ok, d16, GH)
                        pltpu.async_copy(table.at[idx_v], rows_v, sem).wait()
                        pltpu.sync_copy(rows_v, acc_sh.at[didx_v], add=True)

                    plsc.subcore_barrier()
                    pltpu.sync_copy(acc_sh.at[pl.ds(tile * wpt, wpt)],
                                    out.at[cid, pl.ds(base + tile * wpt, wpt)])

                    if rows - 16 * wpt:
                        @pl.when(tile == 15)
                        def _out_tail():
                            pltpu.sync_copy(
                                acc_sh.at[pl.ds(16 * wpt, rows - 16 * wpt)],
                                out.at[cid,
                                       pl.ds(base + 16 * wpt, rows - 16 * wpt)])


    return pl.kernel(
        body,
        out_type=jax.ShapeDtypeStruct((nchunks, N, LANES), jnp.float32),
        mesh=_mesh(),
        scratch_types=[
            pltpu.VMEM((nit, BE), jnp.int32),
            pltpu.VMEM((nit, BE), jnp.int32),
            pltpu.VMEM((BE,), jnp.int32),
            pltpu.VMEM((BE,), jnp.int32),
            pltpu.VMEM((BE, LANES), jnp.float32),
            pltpu.VMEM_SHARED((NACC, LANES), jnp.float32),
            pltpu.SemaphoreType.DMA,
        ],
    )


def _segsum_sc(x, srct, dstt, nit):
    d = x.shape[1]
    nchunks = d // LANES
    table = x.reshape(N, nchunks, LANES).transpose(1, 0, 2).reshape(
        nchunks * N, LANES)
    zeros = jnp.zeros((NACC, LANES), jnp.float32)
    out = _make_scatter_add(nchunks, nit, N)(table, srct, dstt, zeros)
    return out.transpose(1, 0, 2).reshape(N, d)


def _make_gather(nchunks, nit, e_pad):
    m_per_w = e_pad // 32

    def body(table, idxt, out, idx_v, gidx_v, rows_v, sem):
        core = lax.axis_index("c")
        tile = lax.axis_index("s")
        wid = tile * 2 + core
        pltpu.sync_copy(idxt.at[wid], idx_v)
        for c in range(nchunks):
            @pl.loop(0, nit)
            def _step(it):
                for j in range(BE // 16):
                    gidx_v[pl.ds(16 * j, 16)] = (
                        idx_v[it, pl.ds(16 * j, 16)] + c * N)
                pltpu.async_copy(table.at[gidx_v], rows_v, sem).wait()
                pltpu.sync_copy(rows_v,
                                out.at[c, pl.ds(wid * m_per_w + it * BE, BE)])

    return pl.kernel(
        body,
        out_type=jax.ShapeDtypeStruct((nchunks, e_pad, LANES), jnp.float32),
        mesh=_mesh(),
        scratch_types=[
            pltpu.VMEM((nit, BE), jnp.int32),
            pltpu.VMEM((BE,), jnp.int32),
            pltpu.VMEM((BE, LANES), jnp.float32),
            pltpu.SemaphoreType.DMA,
        ],
    )


def _bn(h, g, b):
    mean = h.mean(axis=0)
    var = h.var(axis=0)
    return g * (h - mean) / jnp.sqrt(var + 1e-5) + b


def _gin_conv(x, srct, dstt, nit, W1, b1, g, be, W2, b2):
    agg = _segsum_sc(x, srct, dstt, nit)
    h = x + agg
    h = h @ W1 + b1
    h = _bn(h, g, be)
    h = jax.nn.relu(h)
    return h @ W2 + b2


def _alpha_kernel(gl_ref, gd_ref, att_ref, a_ref, m_ref):
    i = pl.program_id(0)
    parts = []
    for c in range(8):
        z = gl_ref[c] + gd_ref[c]
        e = jnp.maximum(z, 0.2 * z)
        parts.append(jnp.sum(e * att_ref[c][None, :], axis=1))
    a = jnp.stack([parts[0] + parts[1], parts[2] + parts[3],
                   parts[4] + parts[5], parts[6] + parts[7]], axis=1)
    a_ref[...] = a
    m4 = jnp.max(a, axis=0)
    m8 = jnp.concatenate([m4, m4])
    cur = jnp.broadcast_to(m8[:, None], (8, LANES))

    @pl.when(i == 0)
    def _init():
        m_ref[...] = cur

    @pl.when(i > 0)
    def _acc():
        m_ref[...] = jnp.maximum(m_ref[...], cur)


def _scale_kernel(gl_ref, a_ref, m_ref, o_ref):
    c = pl.program_id(0)
    mm = m_ref[0:1, 0:4]
    p = jnp.exp(a_ref[...] - mm)
    lane = lax.broadcasted_iota(jnp.int32, (4, LANES), 1)
    row = lax.broadcasted_iota(jnp.int32, (4, LANES), 0)
    eye = (lane == row).astype(jnp.float32)
    ppad = jax.lax.dot(p, eye)
    hsel = (lax.broadcasted_iota(jnp.int32, (BEB, 4), 1) == c // 2)
    psel = jnp.sum(jnp.where(hsel, p, 0.0), axis=1)
    o_ref[...] = jnp.where(c < 8, gl_ref[0] * psel[:, None], ppad)[None]


def _gat1(h2, s2t, d2t, nit2g, e2_pad, Wl, bl, Wr, br, att, bias):
    xl = h2 @ Wl + bl
    xr = h2 @ Wr + br
    tl = xl.reshape(N, 8, LANES).transpose(1, 0, 2).reshape(8 * N, LANES)
    tr = xr.reshape(N, 8, LANES).transpose(1, 0, 2).reshape(8 * N, LANES)
    gl = _make_gather(8, nit2g, e2_pad)(tl, s2t)
    gd = _make_gather(8, nit2g, e2_pad)(tr, d2t)
    attc = att.reshape(4, 2, LANES).reshape(8, LANES)
    neb = e2_pad // BEB
    a_u, m8 = pl.pallas_call(
        _alpha_kernel,
        grid=(neb,),
        in_specs=[
            pl.BlockSpec((8, BEB, LANES), lambda i: (0, i, 0)),
            pl.BlockSpec((8, BEB, LANES), lambda i: (0, i, 0)),
            pl.BlockSpec((8, LANES), lambda i: (0, 0)),
        ],
        out_specs=[
            pl.BlockSpec((BEB, 4), lambda i: (i, 0)),
            pl.BlockSpec((8, LANES), lambda i: (0, 0)),
        ],
        out_shape=[
            jax.ShapeDtypeStruct((e2_pad, 4), jnp.float32),
            jax.ShapeDtypeStruct((8, LANES), jnp.float32),
        ],
    )(gl, gd, attc)
    m4 = jnp.max(m8, axis=1)[:4]
    marr = jnp.zeros((8, LANES), jnp.float32).at[0, :4].set(m4)
    upd = pl.pallas_call(
        _scale_kernel,
        grid=(9, neb),
        in_specs=[
            pl.BlockSpec((1, BEB, LANES),
                         lambda c, i: (jnp.minimum(c, 7), i, 0)),
            pl.BlockSpec((BEB, 4), lambda c, i: (i, 0)),
            pl.BlockSpec((8, LANES), lambda c, i: (0, 0)),
        ],
        out_specs=pl.BlockSpec((1, BEB, LANES), lambda c, i: (c, i, 0)),
        out_shape=jax.ShapeDtypeStruct((9, e2_pad, LANES), jnp.float32),
    )(gl, a_u, marr)
    iota = jnp.arange(e2_pad, dtype=jnp.int32).reshape(NTILES, -1, BE)
    nit2a = iota.shape[1]
    d2a = d2t.reshape(-1, BE).reshape(NTILES, nit2a, BE)
    zeros = jnp.zeros((NACC, LANES), jnp.float32)
    out9 = _make_scatter_add(9, nit2a, e2_pad)(
        upd.reshape(9 * e2_pad, LANES), iota, d2a, zeros)
    return out9, bias


def _xlr2_kernel(num_ref, den_ref, b1_ref, wl_ref, wr_ref, o_ref):
    den4 = den_ref[0, :, 0:4]
    xl = jnp.zeros((num_ref.shape[1],), jnp.float32)
    xr = xl
    for c in range(8):
        r1c = (num_ref[c] / (den4[:, c // 2:c // 2 + 1] + 1e-16)
               + b1_ref[c][None, :])
        xl = xl + jnp.sum(r1c * wl_ref[c][None, :], axis=1)
        xr = xr + jnp.sum(r1c * wr_ref[c][None, :], axis=1)
    o_ref[...] = jnp.stack([xl, xr, xl, xl, xl, xl, xl, xl], axis=1)


def _ta_kernel(gx_ref, gy_ref, att_ref, a_ref, g_ref, m_ref):
    i = pl.program_id(0)
    z = gx_ref[0, :, 0:16] + gy_ref[0, :, 0:16]
    a = jnp.maximum(z, 0.2 * z) * att_ref[0, 0]
    a_ref[...] = a
    g_ref[...] = gx_ref[0, :, 0:16]
    cur = jnp.broadcast_to(jnp.max(a, axis=0)[None, :], (8, 16))

    @pl.when(i == 0)
    def _init():
        m_ref[...] = cur

    @pl.when(i > 0)
    def _acc():
        m_ref[...] = jnp.maximum(m_ref[...], cur)


def _upd2_kernel(a_ref, g_ref, m_ref, o_ref):
    p = jnp.exp(a_ref[...] - m_ref[0, 0])
    w = p * g_ref[...]
    pb = jnp.broadcast_to(p[:, 0:1], (p.shape[0], LANES))
    wb = jnp.broadcast_to(w[:, 0:1], (p.shape[0], LANES))
    lane = lax.broadcasted_iota(jnp.int32, pb.shape, 1)
    o_ref[...] = jnp.where(lane == 0, pb, jnp.where(lane == 1, wb, 0.0))


def _region_kernel(x_ref, b2_ref, o_ref):
    den = x_ref[:, 0]
    num = x_ref[:, 1]
    res = num / (den + 1e-16) + b2_ref[0, 0]
    z = jnp.zeros((res.shape[0], 15), jnp.float32)
    o_ref[...] = jnp.concatenate([res[:, None], z], axis=1)


def _dot_kernel(a_ref, b_ref, o_ref):
    o_ref[...] = jnp.sum(a_ref[...] * b_ref[...]).reshape(1, 1)


def _big_dot(flat, wd):
    a = flat.reshape(2500, 1024)
    b = wd.reshape(2500, 1024)
    return pl.pallas_call(
        _dot_kernel,
        out_shape=jax.ShapeDtypeStruct((1, 1), jnp.float32),
    )(a, b)


def kernel(eeg_nodes, eeg_idx, W11, b11, g1, be1, W12, b12, W21, b21, g2, be2, W22, b22,
           Wl1, bl1, Wr1, br1, att1, bias1, Wl2, bl2, Wr2, br2, att2, bias2, Wd, bd):
    src = eeg_idx[0].astype(jnp.int32)
    dst = eeg_idx[1].astype(jnp.int32)
    srct, dstt, nit, _ = _pad_edges(src, dst, src.shape[0], NTILES)
    h = _gin_conv(eeg_nodes, srct, dstt, nit, W11, b11, g1, be1, W12, b12)
    h = jax.nn.relu(h)
    h = _gin_conv(h, srct, dstt, nit, W21, b21, g2, be2, W22, b22)

    loop_ids = jnp.arange(N, dtype=jnp.int32)
    s2 = jnp.concatenate([src, loop_ids])
    d2 = jnp.concatenate([dst, loop_ids])
    s2t, d2t, nit2g, e2_pad = _pad_edges(s2, d2, s2.shape[0], 32)

    out9, bias1_ = _gat1(h, s2t, d2t, nit2g, e2_pad, Wl1, bl1, Wr1, br1,
                         att1, bias1)

    b1c = bias1_.reshape(8, LANES)
    wlc = Wl2.reshape(8, LANES)
    wrc = Wr2.reshape(8, LANES)
    BN = 400
    xlr = pl.pallas_call(
        _xlr2_kernel,
        grid=(N // BN,),
        in_specs=[
            pl.BlockSpec((8, BN, LANES), lambda i: (0, i, 0)),
            pl.BlockSpec((1, BN, LANES), lambda i: (8, i, 0)),
            pl.BlockSpec((8, LANES), lambda i: (0, 0)),
            pl.BlockSpec((8, LANES), lambda i: (0, 0)),
            pl.BlockSpec((8, LANES), lambda i: (0, 0)),
        ],
        out_specs=pl.BlockSpec((BN, 8), lambda i: (i, 0)),
        out_shape=jax.ShapeDtypeStruct((N, 8), jnp.float32),
    )(out9, out9, b1c, wlc, wrc)
    xl2 = jnp.concatenate([xlr[:, 0] + bl2[0], jnp.zeros((16,), jnp.float32)])
    xr2 = jnp.concatenate([xlr[:, 1] + br2[0], jnp.zeros((16,), jnp.float32)])

    tl128 = jnp.broadcast_to(xl2[:, None], (NP16, LANES))
    tr128 = jnp.broadcast_to(xr2[:, None], (NP16, LANES))
    gl128 = _make_gather(1, nit2g, e2_pad)(tl128, s2t)
    gr128 = _make_gather(1, nit2g, e2_pad)(tr128, d2t)
    attarr = jnp.full((8, 16), att2[0, 0, 0], jnp.float32)
    a2u16, gl216, m8 = pl.pallas_call(
        _ta_kernel,
        grid=(e2_pad // BEB,),
        in_specs=[
            pl.BlockSpec((1, BEB, LANES), lambda i: (0, i, 0)),
            pl.BlockSpec((1, BEB, LANES), lambda i: (0, i, 0)),
            pl.BlockSpec((8, 16), lambda i: (0, 0)),
        ],
        out_specs=[
            pl.BlockSpec((BEB, 16), lambda i: (i, 0)),
            pl.BlockSpec((BEB, 16), lambda i: (i, 0)),
            pl.BlockSpec((8, 16), lambda i: (0, 0)),
        ],
        out_shape=[
            jax.ShapeDtypeStruct((e2_pad, 16), jnp.float32),
            jax.ShapeDtypeStruct((e2_pad, 16), jnp.float32),
            jax.ShapeDtypeStruct((8, 16), jnp.float32),
        ],
    )(gl128, gr128, attarr)
    m2 = jnp.max(m8)

    m2arr = jnp.full((8, 16), m2, jnp.float32)
    upd128 = pl.pallas_call(
        _upd2_kernel,
        grid=(e2_pad // (8 * BEB),),
        in_specs=[
            pl.BlockSpec((8 * BEB, 16), lambda i: (i, 0)),
            pl.BlockSpec((8 * BEB, 16), lambda i: (i, 0)),
            pl.BlockSpec((8, 16), lambda i: (0, 0)),
        ],
        out_specs=pl.BlockSpec((8 * BEB, LANES), lambda i: (i, 0)),
        out_shape=jax.ShapeDtypeStruct((e2_pad, LANES), jnp.float32),
    )(a2u16, gl216, m2arr)

    iota16 = jnp.arange(e2_pad, dtype=jnp.int32).reshape(NTILES, -1, BE)
    nit2a = iota16.shape[1]
    d2a = d2t.reshape(-1, BE).reshape(NTILES, nit2a, BE)
    zeros = jnp.zeros((NACC, LANES), jnp.float32)
    part2 = _make_scatter_add(1, nit2a, 0)(upd128, iota16, d2a, zeros)

    b2arr = jnp.full((8, LANES), bias2[0], jnp.float32)
    region16 = pl.pallas_call(
        _region_kernel,
        grid=(N // BN,),
        in_specs=[
            pl.BlockSpec((BN, LANES), lambda i: (i, 0)),
            pl.BlockSpec((8, LANES), lambda i: (0, 0)),
        ],
        out_specs=pl.BlockSpec((BN, 16), lambda i: (i, 0)),
        out_shape=jax.ShapeDtypeStruct((N, 16), jnp.float32),
    )(part2[0], b2arr)
    region_scores = region16[:, :1]

    dot = _big_dot(h.reshape(1, N * L)[0], Wd[:, 0])
    dementia_pred = jax.nn.sigmoid(dot + bd)
    return (dementia_pred, region_scores)


# speedup vs baseline: 1.6502x; 1.6502x over previous
import functools

import jax
import jax.numpy as jnp
from jax import lax
from jax.experimental import pallas as pl
from jax.experimental.pallas import tpu as pltpu
from jax.experimental.pallas import tpu_sc as plsc

N = 10000
L = 256
LANES = 128
BE = 256
NTILES = 16
GH = 5120
NACC = GH + 16
ZPT = GH // 16
NG = -(-N // GH)
NP16 = N + 16
BEB = 1024


def _pad_edges(src, dst, n_edges, nworkers):
    step = nworkers * BE
    e_pad = step * ((n_edges + step - 1) // step)
    pad = e_pad - n_edges
    src_p = jnp.concatenate([src, jnp.zeros((pad,), jnp.int32)])
    dst_p = jnp.concatenate([dst, jnp.full((pad,), N, jnp.int32)])
    nit = e_pad // step
    return (src_p.reshape(nworkers, nit, BE), dst_p.reshape(nworkers, nit, BE),
            nit, e_pad)


def _mesh():
    return plsc.VectorSubcoreMesh(core_axis_name="c", subcore_axis_name="s")


def _make_scatter_add(nchunks, nit, stride):
    niter = (nchunks + 1) // 2

    def body(table, srct, dstt, zeros, out, src_v, dst_v, idx_v, didx_v,
             rows_v, acc_sh, sem):
        core = lax.axis_index("c")
        tile = lax.axis_index("s")
        pltpu.sync_copy(srct.at[tile], src_v)
        pltpu.sync_copy(dstt.at[tile], dst_v)
        for ci in range(niter):
            cid = 2 * ci + core
            if 2 * ci + 1 >= nchunks:
                cid = jnp.minimum(cid, nchunks - 1)
            if True:
                for g in range(NG):
                    base = g * GH
                    rows = min(GH, N - base)
                    wpt = (rows // (8 * NTILES)) * 8
                    plsc.subcore_barrier()
                    pltpu.sync_copy(zeros.at[pl.ds(tile * ZPT, ZPT)],
                                    acc_sh.at[pl.ds(tile * ZPT, ZPT)])

                    @pl.when(tile == 0)
                    def _zero_tail():
                        pltpu.sync_copy(
                            zeros.at[pl.ds(16 * ZPT, NACC - 16 * ZPT)],
                            acc_sh.at[pl.ds(16 * ZPT, NACC - 16 * ZPT)])

                    plsc.subcore_barrier()

                    @pl.loop(0, nit)
                    def _edge_step(it):
                        off = cid * stride
                        for j in range(BE // 16):
                            s16 = src_v[it, pl.ds(16 * j, 16)]
                            d16 = dst_v[it, pl.ds(16 * j, 16)] - base
                            ok = (d16 >= 0) & (d16 < rows)
                            idx_v[pl.ds(16 * j, 16)] = s16 + off
                            didx_v[pl.ds(16 * j, 16)] = jnp.where(ok, d16, GH)
                        pltpu.async_copy(table.at[idx_v], rows_v, sem).wait()
                        pltpu.sync_copy(rows_v, acc_sh.at[didx_v], add=True)

                    plsc.subcore_barrier()
                    pltpu.sync_copy(acc_sh.at[pl.ds(tile * wpt, wpt)],
                                    out.at[cid, pl.ds(base + tile * wpt, wpt)])

                    if rows - 16 * wpt:
                        @pl.when(tile == 15)
                        def _out_tail():
                            pltpu.sync_copy(
                                acc_sh.at[pl.ds(16 * wpt, rows - 16 * wpt)],
                                out.at[cid,
                                       pl.ds(base + 16 * wpt, rows - 16 * wpt)])


    return pl.kernel(
        body,
        out_type=jax.ShapeDtypeStruct((nchunks, N, LANES), jnp.float32),
        mesh=_mesh(),
        scratch_types=[
            pltpu.VMEM((nit, BE), jnp.int32),
            pltpu.VMEM((nit, BE), jnp.int32),
            pltpu.VMEM((BE,), jnp.int32),
            pltpu.VMEM((BE,), jnp.int32),
            pltpu.VMEM((BE, LANES), jnp.float32),
            pltpu.VMEM_SHARED((NACC, LANES), jnp.float32),
            pltpu.SemaphoreType.DMA,
        ],
    )


def _segsum_sc(x, srct, dstt, nit):
    d = x.shape[1]
    nchunks = d // LANES
    table = x.reshape(N, nchunks, LANES).transpose(1, 0, 2).reshape(
        nchunks * N, LANES)
    zeros = jnp.zeros((NACC, LANES), jnp.float32)
    out = _make_scatter_add(nchunks, nit, N)(table, srct, dstt, zeros)
    return out.transpose(1, 0, 2).reshape(N, d)


def _make_gather(nchunks, nit, e_pad):
    m_per_w = e_pad // 32

    def body(table, idxt, out, idx_v, gidx_v, rows_v, sem):
        core = lax.axis_index("c")
        tile = lax.axis_index("s")
        wid = tile * 2 + core
        pltpu.sync_copy(idxt.at[wid], idx_v)
        for c in range(nchunks):
            @pl.loop(0, nit)
            def _step(it):
                for j in range(BE // 16):
                    gidx_v[pl.ds(16 * j, 16)] = (
                        idx_v[it, pl.ds(16 * j, 16)] + c * N)
                pltpu.async_copy(table.at[gidx_v], rows_v, sem).wait()
                pltpu.sync_copy(rows_v,
                                out.at[c, pl.ds(wid * m_per_w + it * BE, BE)])

    return pl.kernel(
        body,
        out_type=jax.ShapeDtypeStruct((nchunks, e_pad, LANES), jnp.float32),
        mesh=_mesh(),
        scratch_types=[
            pltpu.VMEM((nit, BE), jnp.int32),
            pltpu.VMEM((BE,), jnp.int32),
            pltpu.VMEM((BE, LANES), jnp.float32),
            pltpu.SemaphoreType.DMA,
        ],
    )


def _bn(h, g, b):
    mean = h.mean(axis=0)
    var = h.var(axis=0)
    return g * (h - mean) / jnp.sqrt(var + 1e-5) + b


def _gin_conv(x, srct, dstt, nit, W1, b1, g, be, W2, b2):
    agg = _segsum_sc(x, srct, dstt, nit)
    h = x + agg
    h = h @ W1 + b1
    h = _bn(h, g, be)
    h = jax.nn.relu(h)
    return h @ W2 + b2


def _alpha_kernel(gl_ref, gd_ref, att_ref, a_ref, m_ref):
    i = pl.program_id(0)
    parts = []
    for c in range(8):
        z = gl_ref[c] + gd_ref[c]
        e = jnp.maximum(z, 0.2 * z)
        parts.append(jnp.sum(e * att_ref[c][None, :], axis=1))
    a = jnp.stack([parts[0] + parts[1], parts[2] + parts[3],
                   parts[4] + parts[5], parts[6] + parts[7]], axis=1)
    a_ref[...] = a
    m4 = jnp.max(a, axis=0)
    m8 = jnp.concatenate([m4, m4])
    cur = jnp.broadcast_to(m8[:, None], (8, LANES))

    @pl.when(i == 0)
    def _init():
        m_ref[...] = cur

    @pl.when(i > 0)
    def _acc():
        m_ref[...] = jnp.maximum(m_ref[...], cur)


def _scale_kernel(gl_ref, a_ref, m_ref, o_ref):
    c = pl.program_id(0)
    mm = m_ref[0:1, 0:4]
    p = jnp.exp(a_ref[...] - mm)
    lane = lax.broadcasted_iota(jnp.int32, (4, LANES), 1)
    row = lax.broadcasted_iota(jnp.int32, (4, LANES), 0)
    eye = (lane == row).astype(jnp.float32)
    ppad = jax.lax.dot(p, eye)
    hsel = (lax.broadcasted_iota(jnp.int32, (BEB, 4), 1) == c // 2)
    psel = jnp.sum(jnp.where(hsel, p, 0.0), axis=1)
    o_ref[...] = jnp.where(c < 8, gl_ref[0] * psel[:, None], ppad)[None]


def _gat1(h2, s2t, d2t, nit2g, e2_pad, Wl, bl, Wr, br, att, bias):
    xl = h2 @ Wl + bl
    xr = h2 @ Wr + br
    tl = xl.reshape(N, 8, LANES).transpose(1, 0, 2).reshape(8 * N, LANES)
    tr = xr.reshape(N, 8, LANES).transpose(1, 0, 2).reshape(8 * N, LANES)
    gl = _make_gather(8, nit2g, e2_pad)(tl, s2t)
    gd = _make_gather(8, nit2g, e2_pad)(tr, d2t)
    attc = att.reshape(4, 2, LANES).reshape(8, LANES)
    neb = e2_pad // BEB
    a_u, m8 = pl.pallas_call(
        _alpha_kernel,
        grid=(neb,),
        in_specs=[
            pl.BlockSpec((8, BEB, LANES), lambda i: (0, i, 0)),
            pl.BlockSpec((8, BEB, LANES), lambda i: (0, i, 0)),
            pl.BlockSpec((8, LANES), lambda i: (0, 0)),
        ],
        out_specs=[
            pl.BlockSpec((BEB, 4), lambda i: (i, 0)),
            pl.BlockSpec((8, LANES), lambda i: (0, 0)),
        ],
        out_shape=[
            jax.ShapeDtypeStruct((e2_pad, 4), jnp.float32),
            jax.ShapeDtypeStruct((8, LANES), jnp.float32),
        ],
    )(gl, gd, attc)
    m4 = jnp.max(m8, axis=1)[:4]
    marr = jnp.zeros((8, LANES), jnp.float32).at[0, :4].set(m4)
    upd = pl.pallas_call(
        _scale_kernel,
        grid=(9, neb),
        in_specs=[
            pl.BlockSpec((1, BEB, LANES),
                         lambda c, i: (jnp.minimum(c, 7), i, 0)),
            pl.BlockSpec((BEB, 4), lambda c, i: (i, 0)),
            pl.BlockSpec((8, LANES), lambda c, i: (0, 0)),
        ],
        out_specs=pl.BlockSpec((1, BEB, LANES), lambda c, i: (c, i, 0)),
        out_shape=jax.ShapeDtypeStruct((9, e2_pad, LANES), jnp.float32),
    )(gl, a_u, marr)
    iota = jnp.arange(e2_pad, dtype=jnp.int32).reshape(NTILES, -1, BE)
    nit2a = iota.shape[1]
    d2a = d2t.reshape(-1, BE).reshape(NTILES, nit2a, BE)
    zeros = jnp.zeros((NACC, LANES), jnp.float32)
    out9 = _make_scatter_add(9, nit2a, e2_pad)(
        upd.reshape(9 * e2_pad, LANES), iota, d2a, zeros)
    return out9, bias


def _xlr2_kernel(num_ref, den_ref, b1_ref, wl_ref, wr_ref, o_ref):
    den4 = den_ref[0, :, 0:4]
    xl = jnp.zeros((num_ref.shape[1],), jnp.float32)
    xr = xl
    for c in range(8):
        r1c = (num_ref[c] / (den4[:, c // 2:c // 2 + 1] + 1e-16)
               + b1_ref[c][None, :])
        xl = xl + jnp.sum(r1c * wl_ref[c][None, :], axis=1)
        xr = xr + jnp.sum(r1c * wr_ref[c][None, :], axis=1)
    o_ref[...] = jnp.stack([xl, xr, xl, xl, xl, xl, xl, xl], axis=1)


def _ta_kernel(gx_ref, gy_ref, att_ref, a_ref, g_ref, m_ref):
    i = pl.program_id(0)
    z = gx_ref[0, :, 0:16] + gy_ref[0, :, 0:16]
    a = jnp.maximum(z, 0.2 * z) * att_ref[0, 0]
    a_ref[...] = a
    g_ref[...] = gx_ref[0, :, 0:16]
    cur = jnp.broadcast_to(jnp.max(a, axis=0)[None, :], (8, 16))

    @pl.when(i == 0)
    def _init():
        m_ref[...] = cur

    @pl.when(i > 0)
    def _acc():
        m_ref[...] = jnp.maximum(m_ref[...], cur)


def _upd2_kernel(a_ref, g_ref, m_ref, o_ref):
    p = jnp.exp(a_ref[...] - m_ref[0, 0])
    w = p * g_ref[...]
    pb = jnp.broadcast_to(p[:, 0:1], (p.shape[0], LANES))
    wb = jnp.broadcast_to(w[:, 0:1], (p.shape[0], LANES))
    lane = lax.broadcasted_iota(jnp.int32, pb.shape, 1)
    o_ref[...] = jnp.where(lane == 0, pb, jnp.where(lane == 1, wb, 0.0))


def _region_kernel(x_ref, b2_ref, o_ref):
    den = x_ref[:, 0]
    num = x_ref[:, 1]
    res = num / (den + 1e-16) + b2_ref[0, 0]
    z = jnp.zeros((res.shape[0], 15), jnp.float32)
    o_ref[...] = jnp.concatenate([res[:, None], z], axis=1)


def _dot_kernel(a_ref, b_ref, o_ref):
    o_ref[...] = jnp.sum(a_ref[...] * b_ref[...]).reshape(1, 1)


def _big_dot(flat, wd):
    a = flat.reshape(2500, 1024)
    b = wd.reshape(2500, 1024)
    return pl.pallas_call(
        _dot_kernel,
        out_shape=jax.ShapeDtypeStruct((1, 1), jnp.float32),
    )(a, b)


def kernel(eeg_nodes, eeg_idx, W11, b11, g1, be1, W12, b12, W21, b21, g2, be2, W22, b22,
           Wl1, bl1, Wr1, br1, att1, bias1, Wl2, bl2, Wr2, br2, att2, bias2, Wd, bd):
    src = eeg_idx[0].astype(jnp.int32)
    dst = eeg_idx[1].astype(jnp.int32)
    srct, dstt, nit, _ = _pad_edges(src, dst, src.shape[0], NTILES)
    h = _gin_conv(eeg_nodes, srct, dstt, nit, W11, b11, g1, be1, W12, b12)
    h = jax.nn.relu(h)
    h = _gin_conv(h, srct, dstt, nit, W21, b21, g2, be2, W22, b22)

    loop_ids = jnp.arange(N, dtype=jnp.int32)
    s2 = jnp.concatenate([src, loop_ids])
    d2 = jnp.concatenate([dst, loop_ids])
    s2t, d2t, nit2g, e2_pad = _pad_edges(s2, d2, s2.shape[0], 32)

    out9, bias1_ = _gat1(h, s2t, d2t, nit2g, e2_pad, Wl1, bl1, Wr1, br1,
                         att1, bias1)

    b1c = bias1_.reshape(8, LANES)
    wlc = Wl2.reshape(8, LANES)
    wrc = Wr2.reshape(8, LANES)
    BN = 400
    xlr = pl.pallas_call(
        _xlr2_kernel,
        grid=(N // BN,),
        in_specs=[
            pl.BlockSpec((8, BN, LANES), lambda i: (0, i, 0)),
            pl.BlockSpec((1, BN, LANES), lambda i: (8, i, 0)),
            pl.BlockSpec((8, LANES), lambda i: (0, 0)),
            pl.BlockSpec((8, LANES), lambda i: (0, 0)),
            pl.BlockSpec((8, LANES), lambda i: (0, 0)),
        ],
        out_specs=pl.BlockSpec((BN, 8), lambda i: (i, 0)),
        out_shape=jax.ShapeDtypeStruct((N, 8), jnp.float32),
    )(out9, out9, b1c, wlc, wrc)
    xl2 = jnp.concatenate([xlr[:, 0] + bl2[0], jnp.zeros((16,), jnp.float32)])
    xr2 = jnp.concatenate([xlr[:, 1] + br2[0], jnp.zeros((16,), jnp.float32)])

    tl128 = jnp.broadcast_to(xl2[:, None], (NP16, LANES))
    tr128 = jnp.broadcast_to(xr2[:, None], (NP16, LANES))
    gl128 = _make_gather(1, nit2g, e2_pad)(tl128, s2t)
    gr128 = _make_gather(1, nit2g, e2_pad)(tr128, d2t)
    attarr = jnp.full((8, 16), att2[0, 0, 0], jnp.float32)
    a2u16, gl216, m8 = pl.pallas_call(
        _ta_kernel,
        grid=(e2_pad // BEB,),
        in_specs=[
            pl.BlockSpec((1, BEB, LANES), lambda i: (0, i, 0)),
            pl.BlockSpec((1, BEB, LANES), lambda i: (0, i, 0)),
            pl.BlockSpec((8, 16), lambda i: (0, 0)),
        ],
        out_specs=[
            pl.BlockSpec((BEB, 16), lambda i: (i, 0)),
            pl.BlockSpec((BEB, 16), lambda i: (i, 0)),
            pl.BlockSpec((8, 16), lambda i: (0, 0)),
        ],
        out_shape=[
            jax.ShapeDtypeStruct((e2_pad, 16), jnp.float32),
            jax.ShapeDtypeStruct((e2_pad, 16), jnp.float32),
            jax.ShapeDtypeStruct((8, 16), jnp.float32),
        ],
    )(gl128, gr128, attarr)
    m2 = jnp.max(m8)

    m2arr = jnp.full((8, 16), m2, jnp.float32)
    upd128 = pl.pallas_call(
        _upd2_kernel,
        grid=(e2_pad // (8 * BEB),),
        in_specs=[
            pl.BlockSpec((8 * BEB, 16), lambda i: (i, 0)),
            pl.BlockSpec((8 * BEB, 16), lambda i: (i, 0)),
            pl.BlockSpec((8, 16), lambda i: (0, 0)),
        ],
        out_specs=pl.BlockSpec((8 * BEB, LANES), lambda i: (i, 0)),
        out_shape=jax.ShapeDtypeStruct((e2_pad, LANES), jnp.float32),
    )(a2u16, gl216, m2arr)

    iota16 = jnp.arange(e2_pad, dtype=jnp.int32).reshape(NTILES, -1, BE)
    nit2a = iota16.shape[1]
    d2a = d2t.reshape(-1, BE).reshape(NTILES, nit2a, BE)
    zeros = jnp.zeros((NACC, LANES), jnp.float32)
    part2 = _make_scatter_add(1, nit2a, 0)(upd128, iota16, d2a, zeros)

    b2arr = jnp.full((8, LANES), bias2[0], jnp.float32)
    region16 = pl.pallas_call(
        _region_kernel,
        grid=(N // BN,),
        in_specs=[
            pl.BlockSpec((BN, LANES), lambda i: (i, 0)),
            pl.BlockSpec((8, LANES), lambda i: (0, 0)),
        ],
        out_specs=pl.BlockSpec((BN, 16), lambda i: (i, 0)),
        out_shape=jax.ShapeDtypeStruct((N, 16), jnp.float32),
    )(part2[0], b2arr)
    region_scores = region16[:, :1]

    dot = _big_dot(h.reshape(1, N * L)[0], Wd[:, 0])
    dementia_pred = jax.nn.sigmoid(dot + bd)
    return (dementia_pred, region_scores)


# speedup vs baseline: 1.6590x; 1.0053x over previous
import jax
import jax.numpy as jnp
from jax import lax
from jax.experimental import pallas as pl
from jax.experimental.pallas import tpu as pltpu
from jax.experimental.pallas import tpu_sc as plsc

N = 10000
L = 256
LANES = 128
BE = 256
NTILES = 16
GH = 5120
NACC = GH + 16
ZPT = GH // 16
NG = -(-N // GH)
NP16 = N + 16
BEB = 1024


def _pad_edges(src, dst, n_edges, nworkers):
    step = nworkers * BE
    e_pad = step * ((n_edges + step - 1) // step)
    pad = e_pad - n_edges
    src_p = jnp.concatenate([src, jnp.zeros((pad,), jnp.int32)])
    dst_p = jnp.concatenate([dst, jnp.full((pad,), N, jnp.int32)])
    nit = e_pad // step
    return (src_p.reshape(nworkers, nit, BE), dst_p.reshape(nworkers, nit, BE),
            nit, e_pad)


def _mesh():
    return plsc.VectorSubcoreMesh(core_axis_name="c", subcore_axis_name="s")


def _make_scatter_add(nchunks, nit, stride):
    niter = (nchunks + 1) // 2

    def body(table, srct, dstt, zeros, out, src_v, dst_v, idx_v, didx_v,
             rows_v, acc_sh, sem):
        core = lax.axis_index("c")
        tile = lax.axis_index("s")
        pltpu.sync_copy(srct.at[tile], src_v)
        pltpu.sync_copy(dstt.at[tile], dst_v)
        for ci in range(niter):
            cid = 2 * ci + core
            if 2 * ci + 1 >= nchunks:
                cid = jnp.minimum(cid, nchunks - 1)
            if True:
                for g in range(NG):
                    base = g * GH
                    rows = min(GH, N - base)
                    wpt = (rows // (8 * NTILES)) * 8
                    plsc.subcore_barrier()
                    pltpu.sync_copy(zeros.at[pl.ds(tile * ZPT, ZPT)],
                                    acc_sh.at[pl.ds(tile * ZPT, ZPT)])

                    @pl.when(tile == 0)
                    def _zero_tail():
                        pltpu.sync_copy(
                            zeros.at[pl.ds(16 * ZPT, NACC - 16 * ZPT)],
                            acc_sh.at[pl.ds(16 * ZPT, NACC - 16 * ZPT)])

                    plsc.subcore_barrier()

                    @pl.loop(0, nit)
                    def _edge_step(it):
                        off = cid * stride
                        for j in range(BE // 16):
                            s16 = src_v[it, pl.ds(16 * j, 16)]
                            d16 = dst_v[it, pl.ds(16 * j, 16)] - base
                            ok = (d16 >= 0) & (d16 < rows)
                            idx_v[pl.ds(16 * j, 16)] = s16 + off
                            didx_v[pl.ds(16 * j, 16)] = jnp.where(ok, d16, GH)
                        pltpu.async_copy(table.at[idx_v], rows_v, sem).wait()
                        pltpu.sync_copy(rows_v, acc_sh.at[didx_v], add=True)

                    plsc.subcore_barrier()
                    pltpu.sync_copy(acc_sh.at[pl.ds(tile * wpt, wpt)],
                                    out.at[cid, pl.ds(base + tile * wpt, wpt)])

                    if rows - 16 * wpt:
                        @pl.when(tile == 15)
                        def _out_tail():
                            pltpu.sync_copy(
                                acc_sh.at[pl.ds(16 * wpt, rows - 16 * wpt)],
                                out.at[cid,
                                       pl.ds(base + 16 * wpt, rows - 16 * wpt)])


    return pl.kernel(
        body,
        out_type=jax.ShapeDtypeStruct((nchunks, N, LANES), jnp.float32),
        mesh=_mesh(),
        scratch_types=[
            pltpu.VMEM((nit, BE), jnp.int32),
            pltpu.VMEM((nit, BE), jnp.int32),
            pltpu.VMEM((BE,), jnp.int32),
            pltpu.VMEM((BE,), jnp.int32),
            pltpu.VMEM((BE, LANES), jnp.float32),
            pltpu.VMEM_SHARED((NACC, LANES), jnp.float32),
            pltpu.SemaphoreType.DMA,
        ],
    )


def _segsum_sc(x, srct, dstt, nit):
    d = x.shape[1]
    nchunks = d // LANES
    table = x.reshape(N, nchunks, LANES).transpose(1, 0, 2).reshape(
        nchunks * N, LANES)
    zeros = jnp.zeros((NACC, LANES), jnp.float32)
    out = _make_scatter_add(nchunks, nit, N)(table, srct, dstt, zeros)
    return out.transpose(1, 0, 2).reshape(N, d)


def _make_gather(nchunks, nit, e_pad):
    m_per_w = e_pad // 32

    def body(table, idxt, out, idx_v, gidx_v, rows_v, sem):
        core = lax.axis_index("c")
        tile = lax.axis_index("s")
        wid = tile * 2 + core
        pltpu.sync_copy(idxt.at[wid], idx_v)
        for c in range(nchunks):
            @pl.loop(0, nit)
            def _step(it):
                for j in range(BE // 16):
                    gidx_v[pl.ds(16 * j, 16)] = (
                        idx_v[it, pl.ds(16 * j, 16)] + c * N)
                pltpu.async_copy(table.at[gidx_v], rows_v, sem).wait()
                pltpu.sync_copy(rows_v,
                                out.at[c, pl.ds(wid * m_per_w + it * BE, BE)])

    return pl.kernel(
        body,
        out_type=jax.ShapeDtypeStruct((nchunks, e_pad, LANES), jnp.float32),
        mesh=_mesh(),
        scratch_types=[
            pltpu.VMEM((nit, BE), jnp.int32),
            pltpu.VMEM((BE,), jnp.int32),
            pltpu.VMEM((BE, LANES), jnp.float32),
            pltpu.SemaphoreType.DMA,
        ],
    )


def _bn(h, g, b):
    mean = h.mean(axis=0)
    var = h.var(axis=0)
    return g * (h - mean) / jnp.sqrt(var + 1e-5) + b


def _gin_conv(x, srct, dstt, nit, W1, b1, g, be, W2, b2):
    agg = _segsum_sc(x, srct, dstt, nit)
    h = x + agg
    h = h @ W1 + b1
    h = _bn(h, g, be)
    h = jax.nn.relu(h)
    return h @ W2 + b2


def _alpha_kernel(gl_ref, gd_ref, att_ref, a_ref, m_ref):
    i = pl.program_id(0)
    parts = []
    for c in range(8):
        z = gl_ref[c] + gd_ref[c]
        e = jnp.maximum(z, 0.2 * z)
        parts.append(jnp.sum(e * att_ref[c][None, :], axis=1))
    a = jnp.stack([parts[0] + parts[1], parts[2] + parts[3],
                   parts[4] + parts[5], parts[6] + parts[7]], axis=1)
    a_ref[...] = a
    m4 = jnp.max(a, axis=0)
    m8 = jnp.concatenate([m4, m4])
    cur = jnp.broadcast_to(m8[:, None], (8, LANES))

    @pl.when(i == 0)
    def _init():
        m_ref[...] = cur

    @pl.when(i > 0)
    def _acc():
        m_ref[...] = jnp.maximum(m_ref[...], cur)


def _scale_kernel(gl_ref, a_ref, m_ref, o_ref):
    c = pl.program_id(0)
    mm = m_ref[0:1, 0:4]
    p = jnp.exp(a_ref[...] - mm)
    lane = lax.broadcasted_iota(jnp.int32, (4, LANES), 1)
    row = lax.broadcasted_iota(jnp.int32, (4, LANES), 0)
    eye = (lane == row).astype(jnp.float32)
    ppad = jax.lax.dot(p, eye)
    hsel = (lax.broadcasted_iota(jnp.int32, (BEB, 4), 1) == c // 2)
    psel = jnp.sum(jnp.where(hsel, p, 0.0), axis=1)
    o_ref[...] = jnp.where(c < 8, gl_ref[0] * psel[:, None], ppad)[None]


def _gat1(h2, s2t, d2t, nit2g, e2_pad, Wl, bl, Wr, br, att, bias):
    xl = h2 @ Wl + bl
    xr = h2 @ Wr + br
    tl = xl.reshape(N, 8, LANES).transpose(1, 0, 2).reshape(8 * N, LANES)
    tr = xr.reshape(N, 8, LANES).transpose(1, 0, 2).reshape(8 * N, LANES)
    gl = _make_gather(8, nit2g, e2_pad)(tl, s2t)
    gd = _make_gather(8, nit2g, e2_pad)(tr, d2t)
    attc = att.reshape(4, 2, LANES).reshape(8, LANES)
    neb = e2_pad // BEB
    a_u, m8 = pl.pallas_call(
        _alpha_kernel,
        grid=(neb,),
        in_specs=[
            pl.BlockSpec((8, BEB, LANES), lambda i: (0, i, 0)),
            pl.BlockSpec((8, BEB, LANES), lambda i: (0, i, 0)),
            pl.BlockSpec((8, LANES), lambda i: (0, 0)),
        ],
        out_specs=[
            pl.BlockSpec((BEB, 4), lambda i: (i, 0)),
            pl.BlockSpec((8, LANES), lambda i: (0, 0)),
        ],
        out_shape=[
            jax.ShapeDtypeStruct((e2_pad, 4), jnp.float32),
            jax.ShapeDtypeStruct((8, LANES), jnp.float32),
        ],
    )(gl, gd, attc)
    m4 = jnp.max(m8, axis=1)[:4]
    marr = jnp.zeros((8, LANES), jnp.float32).at[0, :4].set(m4)
    upd = pl.pallas_call(
        _scale_kernel,
        grid=(9, neb),
        in_specs=[
            pl.BlockSpec((1, BEB, LANES),
                         lambda c, i: (jnp.minimum(c, 7), i, 0)),
            pl.BlockSpec((BEB, 4), lambda c, i: (i, 0)),
            pl.BlockSpec((8, LANES), lambda c, i: (0, 0)),
        ],
        out_specs=pl.BlockSpec((1, BEB, LANES), lambda c, i: (c, i, 0)),
        out_shape=jax.ShapeDtypeStruct((9, e2_pad, LANES), jnp.float32),
    )(gl, a_u, marr)
    iota = jnp.arange(e2_pad, dtype=jnp.int32).reshape(NTILES, -1, BE)
    nit2a = iota.shape[1]
    d2a = d2t.reshape(-1, BE).reshape(NTILES, nit2a, BE)
    zeros = jnp.zeros((NACC, LANES), jnp.float32)
    out9 = _make_scatter_add(9, nit2a, e2_pad)(
        upd.reshape(9 * e2_pad, LANES), iota, d2a, zeros)
    return out9, bias


def _xlr2_kernel(num_ref, den_ref, b1_ref, wl_ref, wr_ref, o_ref):
    den4 = den_ref[0, :, 0:4]
    xl = jnp.zeros((num_ref.shape[1],), jnp.float32)
    xr = xl
    for c in range(8):
        r1c = (num_ref[c] / (den4[:, c // 2:c // 2 + 1] + 1e-16)
               + b1_ref[c][None, :])
        xl = xl + jnp.sum(r1c * wl_ref[c][None, :], axis=1)
        xr = xr + jnp.sum(r1c * wr_ref[c][None, :], axis=1)
    o_ref[...] = jnp.stack([xl, xr, xl, xl, xl, xl, xl, xl], axis=1)


def _ta_kernel(gx_ref, gy_ref, att_ref, a_ref, g_ref, m_ref):
    i = pl.program_id(0)
    z = gx_ref[0, :, 0:16] + gy_ref[0, :, 0:16]
    a = jnp.maximum(z, 0.2 * z) * att_ref[0, 0]
    a_ref[...] = a
    g_ref[...] = gx_ref[0, :, 0:16]
    cur = jnp.broadcast_to(jnp.max(a, axis=0)[None, :], (8, 16))

    @pl.when(i == 0)
    def _init():
        m_ref[...] = cur

    @pl.when(i > 0)
    def _acc():
        m_ref[...] = jnp.maximum(m_ref[...], cur)


def _upd2_kernel(a_ref, g_ref, m_ref, o_ref):
    p = jnp.exp(a_ref[...] - m_ref[0, 0])
    w = p * g_ref[...]
    pb = jnp.broadcast_to(p[:, 0:1], (p.shape[0], LANES))
    wb = jnp.broadcast_to(w[:, 0:1], (p.shape[0], LANES))
    lane = lax.broadcasted_iota(jnp.int32, pb.shape, 1)
    o_ref[...] = jnp.where(lane == 0, pb, jnp.where(lane == 1, wb, 0.0))


def _region_kernel(x_ref, b2_ref, o_ref):
    den = x_ref[:, 0]
    num = x_ref[:, 1]
    res = num / (den + 1e-16) + b2_ref[0, 0]
    z = jnp.zeros((res.shape[0], 15), jnp.float32)
    o_ref[...] = jnp.concatenate([res[:, None], z], axis=1)


def _dot_kernel(a_ref, b_ref, o_ref):
    o_ref[...] = jnp.sum(a_ref[...] * b_ref[...]).reshape(1, 1)


def _big_dot(flat, wd):
    a = flat.reshape(2500, 1024)
    b = wd.reshape(2500, 1024)
    return pl.pallas_call(
        _dot_kernel,
        out_shape=jax.ShapeDtypeStruct((1, 1), jnp.float32),
    )(a, b)


def kernel(eeg_nodes, eeg_idx, W11, b11, g1, be1, W12, b12, W21, b21, g2, be2, W22, b22,
           Wl1, bl1, Wr1, br1, att1, bias1, Wl2, bl2, Wr2, br2, att2, bias2, Wd, bd):
    src = eeg_idx[0].astype(jnp.int32)
    dst = eeg_idx[1].astype(jnp.int32)
    srct, dstt, nit, _ = _pad_edges(src, dst, src.shape[0], NTILES)
    h = _gin_conv(eeg_nodes, srct, dstt, nit, W11, b11, g1, be1, W12, b12)
    h = jax.nn.relu(h)
    h = _gin_conv(h, srct, dstt, nit, W21, b21, g2, be2, W22, b22)

    loop_ids = jnp.arange(N, dtype=jnp.int32)
    s2 = jnp.concatenate([src, loop_ids])
    d2 = jnp.concatenate([dst, loop_ids])
    s2t, d2t, nit2g, e2_pad = _pad_edges(s2, d2, s2.shape[0], 32)

    out9, bias1_ = _gat1(h, s2t, d2t, nit2g, e2_pad, Wl1, bl1, Wr1, br1,
                         att1, bias1)

    b1c = bias1_.reshape(8, LANES)
    wlc = Wl2.reshape(8, LANES)
    wrc = Wr2.reshape(8, LANES)
    BN = 400
    xlr = pl.pallas_call(
        _xlr2_kernel,
        grid=(N // BN,),
        in_specs=[
            pl.BlockSpec((8, BN, LANES), lambda i: (0, i, 0)),
            pl.BlockSpec((1, BN, LANES), lambda i: (8, i, 0)),
            pl.BlockSpec((8, LANES), lambda i: (0, 0)),
            pl.BlockSpec((8, LANES), lambda i: (0, 0)),
            pl.BlockSpec((8, LANES), lambda i: (0, 0)),
        ],
        out_specs=pl.BlockSpec((BN, 8), lambda i: (i, 0)),
        out_shape=jax.ShapeDtypeStruct((N, 8), jnp.float32),
    )(out9, out9, b1c, wlc, wrc)
    xl2 = jnp.concatenate([xlr[:, 0] + bl2[0], jnp.zeros((16,), jnp.float32)])
    xr2 = jnp.concatenate([xlr[:, 1] + br2[0], jnp.zeros((16,), jnp.float32)])

    tl128 = jnp.broadcast_to(xl2[:, None], (NP16, LANES))
    tr128 = jnp.broadcast_to(xr2[:, None], (NP16, LANES))
    gl128 = _make_gather(1, nit2g, e2_pad)(tl128, s2t)
    gr128 = _make_gather(1, nit2g, e2_pad)(tr128, d2t)
    attarr = jnp.full((8, 16), att2[0, 0, 0], jnp.float32)
    a2u16, gl216, m8 = pl.pallas_call(
        _ta_kernel,
        grid=(e2_pad // BEB,),
        in_specs=[
            pl.BlockSpec((1, BEB, LANES), lambda i: (0, i, 0)),
            pl.BlockSpec((1, BEB, LANES), lambda i: (0, i, 0)),
            pl.BlockSpec((8, 16), lambda i: (0, 0)),
        ],
        out_specs=[
            pl.BlockSpec((BEB, 16), lambda i: (i, 0)),
            pl.BlockSpec((BEB, 16), lambda i: (i, 0)),
            pl.BlockSpec((8, 16), lambda i: (0, 0)),
        ],
        out_shape=[
            jax.ShapeDtypeStruct((e2_pad, 16), jnp.float32),
            jax.ShapeDtypeStruct((e2_pad, 16), jnp.float32),
            jax.ShapeDtypeStruct((8, 16), jnp.float32),
        ],
    )(gl128, gr128, attarr)
    m2 = jnp.max(m8)

    m2arr = jnp.full((8, 16), m2, jnp.float32)
    upd128 = pl.pallas_call(
        _upd2_kernel,
        grid=(e2_pad // (8 * BEB),),
        in_specs=[
            pl.BlockSpec((8 * BEB, 16), lambda i: (i, 0)),
            pl.BlockSpec((8 * BEB, 16), lambda i: (i, 0)),
            pl.BlockSpec((8, 16), lambda i: (0, 0)),
        ],
        out_specs=pl.BlockSpec((8 * BEB, LANES), lambda i: (i, 0)),
        out_shape=jax.ShapeDtypeStruct((e2_pad, LANES), jnp.float32),
    )(a2u16, gl216, m2arr)

    iota16 = jnp.arange(e2_pad, dtype=jnp.int32).reshape(NTILES, -1, BE)
    nit2a = iota16.shape[1]
    d2a = d2t.reshape(-1, BE).reshape(NTILES, nit2a, BE)
    zeros = jnp.zeros((NACC, LANES), jnp.float32)
    part2 = _make_scatter_add(1, nit2a, 0)(upd128, iota16, d2a, zeros)

    b2arr = jnp.full((8, LANES), bias2[0], jnp.float32)
    region16 = pl.pallas_call(
        _region_kernel,
        grid=(N // BN,),
        in_specs=[
            pl.BlockSpec((BN, LANES), lambda i: (i, 0)),
            pl.BlockSpec((8, LANES), lambda i: (0, 0)),
        ],
        out_specs=pl.BlockSpec((BN, 16), lambda i: (i, 0)),
        out_shape=jax.ShapeDtypeStruct((N, 16), jnp.float32),
    )(part2[0], b2arr)
    region_scores = region16[:, :1]

    dot = _big_dot(h.reshape(1, N * L)[0], Wd[:, 0])
    dementia_pred = jax.nn.sigmoid(dot + bd)
    return (dementia_pred, region_scores)
